# baseline (speedup 1.0000x reference)
import functools

import jax
import jax.numpy as jnp
from jax import lax
from jax.experimental import pallas as pl
from jax.experimental.pallas import tpu as pltpu
from jax.experimental.pallas import tpu_sc as plsc

_B = 32
_NB = 8732
_C = 21
_M = 50
_N = _B * _NB
_ROWS = _N // 128
_P = 4096
_NWORK = 32
_PW = _P // _NWORK
_BLK = 2048


def _lse_body(conf_ref, score_ref, lse_ref):
    x = conf_ref[...]
    m = jnp.max(x, axis=0, keepdims=True)
    s = jnp.sum(jnp.exp(x - m), axis=0, keepdims=True)
    lse = m + jnp.log(s)
    score_ref[...] = x[0:1, :] - lse
    lse_ref[...] = lse


def _orderable(x):
    i = lax.bitcast_convert_type(x, jnp.int32)
    return i ^ ((i >> 31) & jnp.int32(0x7FFFFFFF))


def _loss_body(score_ref, sg_ref, lg_ref, cg_ref, wv_ref, wp_ref,
               bb_ref, pb_ref, tb_ref, closs_ref, bloss_ref):
    wv = wv_ref[...]
    wp = wp_ref[...]
    pnum = jnp.sum(wp)
    pos_sum = jnp.sum(wp * (cg_ref[...] - lg_ref[...]))

    px1, py1, px2, py2 = pb_ref[0], pb_ref[1], pb_ref[2], pb_ref[3]
    tx1, ty1, tx2, ty2 = tb_ref[0], tb_ref[1], tb_ref[2], tb_ref[3]
    pw = px2 - px1
    ph = py2 - py1
    enc = ((tx1 + tx2) * 0.5 - (px1 + px2) * 0.5) / pw, \
          ((ty1 + ty2) * 0.5 - (py1 + py2) * 0.5) / ph, \
          jnp.log((tx2 - tx1) / pw), \
          jnp.log((ty2 - ty1) / ph)
    bsum = jnp.float32(0.0)
    for d in range(4):
        diff = bb_ref[d] - enc[d]
        ad = jnp.abs(diff)
        sl1 = jnp.where(ad < 1.0, 0.5 * diff * diff, ad - 0.5)
        bsum = bsum + jnp.sum(wv * sl1)
    bcount = 4.0 * jnp.sum(wv)

    sc = score_ref[...]
    skey = _orderable(sc)
    mskey = jnp.where(wv > 0.0, _orderable(sg_ref[...]),
                      jnp.int32(0x7FFFFFFF))
    t_i = (3.0 * pnum).astype(jnp.int32)
    sign = jnp.int32(-0x80000000)

    p = jnp.int32(0)
    for it in range(32):
        b = 31 - it
        c = p | (jnp.int32(1) << b)
        cs = c ^ sign
        cnt = (jnp.sum((skey < cs).astype(jnp.int32))
               - jnp.sum((mskey < cs).astype(jnp.int32)))
        p = jnp.where(cnt < t_i, c, p)
    vkey = p ^ sign
    vi = jnp.where(vkey >= 0, vkey, vkey ^ jnp.int32(0x7FFFFFFF))
    vf = lax.bitcast_convert_type(vi, jnp.float32)
    lt = skey < vkey
    mlt = mskey < vkey
    cnt_lt = (jnp.sum(lt.astype(jnp.float32))
              - jnp.sum(mlt.astype(jnp.float32)))
    sum_lt = (jnp.sum(jnp.where(lt, sc, 0.0))
              - jnp.sum(jnp.where(mlt, sg_ref[...], 0.0)))
    t_f = t_i.astype(jnp.float32)
    neg_sum = sum_lt + (t_f - cnt_lt) * vf

    closs_ref[0, 0] = -(pos_sum + neg_sum) / (pnum + t_f)
    bloss_ref[0, 0] = bsum / bcount


def _sc_gather(idx_hbm, score_hbm, lse_hbm, cflat_hbm, bbfl_hbm, pdfl_hbm,
               out_hbm, idx_v, dat_v, sem):
    wid = lax.axis_index("s") * 2 + lax.axis_index("c")
    base = wid * _PW
    tables = (score_hbm, lse_hbm, cflat_hbm,
              bbfl_hbm, bbfl_hbm, bbfl_hbm, bbfl_hbm,
              pdfl_hbm, pdfl_hbm, pdfl_hbm, pdfl_hbm)
    pltpu.sync_copy(idx_hbm.at[:, pl.ds(base, _PW)], idx_v)
    copies = [pltpu.async_copy(tab.at[idx_v.at[j]], dat_v.at[j], sem)
              for j, tab in enumerate(tables)]
    for c in copies:
        c.wait()
    pltpu.sync_copy(dat_v, out_hbm.at[:, pl.ds(base, _PW)])


def kernel(confOut, bboxOut, target, predBoxes):
    num = target[:, 0].astype(jnp.int32)
    rest = target[:, 1:1 + 6 * _M].reshape(_B, _M, 6)
    cls = rest[:, :, 0].astype(jnp.int32)
    tbx = rest[:, :, 1:5]
    kk = jnp.clip(rest[:, :, 5].astype(jnp.int32), 0, _NB - 1)
    valid = jnp.arange(_M)[None, :] < num[:, None]
    posm = valid & (cls > 0)
    fi = jnp.where(valid, jnp.arange(_B)[:, None] * _NB + kk, 0)
    ci = jnp.where(posm, fi * _C + jnp.clip(cls, 0, _C - 1), 0)
    kidx = jnp.where(valid, kk, 0)
    wv = valid.astype(jnp.float32)
    wp = posm.astype(jnp.float32)
    safe = jnp.array([0.0, 0.0, 1.0, 1.0], jnp.float32)
    tbs = jnp.where(valid[:, :, None], tbx, safe)

    pad = _P - _B * _M
    fi_p = jnp.concatenate([fi.reshape(-1), jnp.zeros(pad, jnp.int32)])
    ci_p = jnp.concatenate([ci.reshape(-1), jnp.zeros(pad, jnp.int32)])
    kk_p = jnp.concatenate([kidx.reshape(-1), jnp.zeros(pad, jnp.int32)])
    wv_p = jnp.concatenate([wv.reshape(-1), jnp.zeros(pad, jnp.float32)])
    wp_p = jnp.concatenate([wp.reshape(-1), jnp.zeros(pad, jnp.float32)])
    tb_p = jnp.concatenate(
        [tbs.reshape(-1, 4), jnp.broadcast_to(safe, (pad, 4))])

    fi4 = fi_p * 4
    pk4 = kk_p * 4
    idx_all = jnp.stack([fi_p, fi_p, ci_p,
                         fi4, fi4 + 1, fi4 + 2, fi4 + 3,
                         pk4, pk4 + 1, pk4 + 2, pk4 + 3])

    conf_t = confOut.reshape(_N, _C).T
    cflat = confOut.reshape(_N * _C)
    bbfl = bboxOut.reshape(_N * 4)
    pdfl = predBoxes.reshape(_NB * 4)

    grid_a = (_N + _BLK - 1) // _BLK
    score0, lse = pl.pallas_call(
        _lse_body,
        grid=(grid_a,),
        in_specs=[pl.BlockSpec((_C, _BLK), lambda i: (0, i))],
        out_specs=[pl.BlockSpec((1, _BLK), lambda i: (0, i)),
                   pl.BlockSpec((1, _BLK), lambda i: (0, i))],
        out_shape=[jax.ShapeDtypeStruct((1, _N), jnp.float32),
                   jax.ShapeDtypeStruct((1, _N), jnp.float32)],
    )(conf_t)

    score_flat = score0.reshape(_N)
    lse_flat = lse.reshape(_N)

    mesh = plsc.VectorSubcoreMesh(core_axis_name="c", subcore_axis_name="s",
                                  num_cores=2, num_subcores=16)
    gath = pl.kernel(
        _sc_gather,
        out_type=jax.ShapeDtypeStruct((11, _P), jnp.float32),
        mesh=mesh,
        scratch_types=[pltpu.VMEM((11, _PW), jnp.int32),
                       pltpu.VMEM((11, _PW), jnp.float32),
                       pltpu.SemaphoreType.DMA],
    )(idx_all, score_flat, lse_flat, cflat, bbfl, pdfl)

    shp = (_P // 128, 128)
    closs, bloss = pl.pallas_call(
        _loss_body,
        out_specs=[pl.BlockSpec(memory_space=pltpu.SMEM),
                   pl.BlockSpec(memory_space=pltpu.SMEM)],
        out_shape=[jax.ShapeDtypeStruct((1, 1), jnp.float32),
                   jax.ShapeDtypeStruct((1, 1), jnp.float32)],
    )(score0.reshape(_ROWS, 128), gath[0].reshape(shp),
      gath[1].reshape(shp), gath[2].reshape(shp),
      wv_p.reshape(shp), wp_p.reshape(shp),
      gath[3:7].reshape(4, *shp), gath[7:11].reshape(4, *shp),
      tb_p.T.reshape(4, *shp))

    return (closs[0, 0], bloss[0, 0])


# speedup vs baseline: 1.0361x; 1.0361x over previous
import functools

import jax
import jax.numpy as jnp
from jax import lax
from jax.experimental import pallas as pl
from jax.experimental.pallas import tpu as pltpu
from jax.experimental.pallas import tpu_sc as plsc

_B = 32
_NB = 8732
_C = 21
_M = 50
_N = _B * _NB
_ROWS = _N // 128
_P = 2048
_NWORK = 32
_PW = _P // _NWORK
_BLK = 2048


def _lse_body(conf_ref, score_ref, lse_ref):
    x = conf_ref[...]
    m = jnp.max(x, axis=0, keepdims=True)
    s = jnp.sum(jnp.exp(x - m), axis=0, keepdims=True)
    lse = m + jnp.log(s)
    score_ref[...] = x[0:1, :] - lse
    lse_ref[...] = lse


def _orderable(x):
    i = lax.bitcast_convert_type(x, jnp.int32)
    return i ^ ((i >> 31) & jnp.int32(0x7FFFFFFF))


def _loss_body(score_ref, sg_ref, lg_ref, cg_ref, wv_ref, wp_ref,
               bb_ref, pb_ref, tb_ref, closs_ref, bloss_ref):
    wv = wv_ref[...]
    wp = wp_ref[...]
    pnum = jnp.sum(wp)
    pos_sum = jnp.sum(wp * (cg_ref[...] - lg_ref[...]))

    px1, py1, px2, py2 = pb_ref[0], pb_ref[1], pb_ref[2], pb_ref[3]
    tx1, ty1, tx2, ty2 = tb_ref[0], tb_ref[1], tb_ref[2], tb_ref[3]
    pw = px2 - px1
    ph = py2 - py1
    enc = ((tx1 + tx2) * 0.5 - (px1 + px2) * 0.5) / pw, \
          ((ty1 + ty2) * 0.5 - (py1 + py2) * 0.5) / ph, \
          jnp.log((tx2 - tx1) / pw), \
          jnp.log((ty2 - ty1) / ph)
    bsum = jnp.float32(0.0)
    for d in range(4):
        diff = bb_ref[d] - enc[d]
        ad = jnp.abs(diff)
        sl1 = jnp.where(ad < 1.0, 0.5 * diff * diff, ad - 0.5)
        bsum = bsum + jnp.sum(wv * sl1)
    bcount = 4.0 * jnp.sum(wv)

    sc = score_ref[...]
    skey = _orderable(sc)
    mskey = jnp.where(wv > 0.0, _orderable(sg_ref[...]),
                      jnp.int32(0x7FFFFFFF))
    t_i = (3.0 * pnum).astype(jnp.int32)
    sign = jnp.int32(-0x80000000)

    def step(it, p):
        b = 31 - it
        c = p | (jnp.int32(1) << b)
        cs = c ^ sign
        cnt = (jnp.sum((skey < cs).astype(jnp.int32))
               - jnp.sum((mskey < cs).astype(jnp.int32)))
        return jnp.where(cnt < t_i, c, p)

    p = lax.fori_loop(0, 32, step, jnp.int32(0))
    vkey = p ^ sign
    vi = jnp.where(vkey >= 0, vkey, vkey ^ jnp.int32(0x7FFFFFFF))
    vf = lax.bitcast_convert_type(vi, jnp.float32)
    lt = skey < vkey
    mlt = mskey < vkey
    cnt_lt = (jnp.sum(lt.astype(jnp.float32))
              - jnp.sum(mlt.astype(jnp.float32)))
    sum_lt = (jnp.sum(jnp.where(lt, sc, 0.0))
              - jnp.sum(jnp.where(mlt, sg_ref[...], 0.0)))
    t_f = t_i.astype(jnp.float32)
    neg_sum = sum_lt + (t_f - cnt_lt) * vf

    closs_ref[0, 0] = -(pos_sum + neg_sum) / (pnum + t_f)
    bloss_ref[0, 0] = bsum / bcount


def _sc_gather(idx_hbm, score_hbm, lse_hbm, cflat_hbm, bbfl_hbm, pdfl_hbm,
               out_hbm, idx_v, dat_v, sem):
    wid = lax.axis_index("s") * 2 + lax.axis_index("c")
    base = wid * _PW
    tables = (score_hbm, lse_hbm, cflat_hbm,
              bbfl_hbm, bbfl_hbm, bbfl_hbm, bbfl_hbm,
              pdfl_hbm, pdfl_hbm, pdfl_hbm, pdfl_hbm)
    for j, tab in enumerate(tables):
        pltpu.sync_copy(idx_hbm.at[j, pl.ds(base, _PW)], idx_v)
        pltpu.async_copy(tab.at[idx_v], dat_v, sem).wait()
        pltpu.sync_copy(dat_v, out_hbm.at[j, pl.ds(base, _PW)])


def kernel(confOut, bboxOut, target, predBoxes):
    num = target[:, 0].astype(jnp.int32)
    rest = target[:, 1:1 + 6 * _M].reshape(_B, _M, 6)
    cls = rest[:, :, 0].astype(jnp.int32)
    tbx = rest[:, :, 1:5]
    kk = jnp.clip(rest[:, :, 5].astype(jnp.int32), 0, _NB - 1)
    valid = jnp.arange(_M)[None, :] < num[:, None]
    posm = valid & (cls > 0)
    fi = jnp.where(valid, jnp.arange(_B)[:, None] * _NB + kk, 0)
    ci = jnp.where(posm, fi * _C + jnp.clip(cls, 0, _C - 1), 0)
    kidx = jnp.where(valid, kk, 0)
    wv = valid.astype(jnp.float32)
    wp = posm.astype(jnp.float32)
    safe = jnp.array([0.0, 0.0, 1.0, 1.0], jnp.float32)
    tbs = jnp.where(valid[:, :, None], tbx, safe)

    pad = _P - _B * _M
    fi_p = jnp.concatenate([fi.reshape(-1), jnp.zeros(pad, jnp.int32)])
    ci_p = jnp.concatenate([ci.reshape(-1), jnp.zeros(pad, jnp.int32)])
    kk_p = jnp.concatenate([kidx.reshape(-1), jnp.zeros(pad, jnp.int32)])
    wv_p = jnp.concatenate([wv.reshape(-1), jnp.zeros(pad, jnp.float32)])
    wp_p = jnp.concatenate([wp.reshape(-1), jnp.zeros(pad, jnp.float32)])
    tb_p = jnp.concatenate(
        [tbs.reshape(-1, 4), jnp.broadcast_to(safe, (pad, 4))])

    fi4 = fi_p * 4
    pk4 = kk_p * 4
    idx_all = jnp.stack([fi_p, fi_p, ci_p,
                         fi4, fi4 + 1, fi4 + 2, fi4 + 3,
                         pk4, pk4 + 1, pk4 + 2, pk4 + 3])

    conf_t = confOut.reshape(_N, _C).T
    cflat = confOut.reshape(_N * _C)
    bbfl = bboxOut.reshape(_N * 4)
    pdfl = predBoxes.reshape(_NB * 4)

    grid_a = (_N + _BLK - 1) // _BLK
    score0, lse = pl.pallas_call(
        _lse_body,
        grid=(grid_a,),
        in_specs=[pl.BlockSpec((_C, _BLK), lambda i: (0, i))],
        out_specs=[pl.BlockSpec((1, _BLK), lambda i: (0, i)),
                   pl.BlockSpec((1, _BLK), lambda i: (0, i))],
        out_shape=[jax.ShapeDtypeStruct((1, _N), jnp.float32),
                   jax.ShapeDtypeStruct((1, _N), jnp.float32)],
    )(conf_t)

    score_flat = score0.reshape(_N)
    lse_flat = lse.reshape(_N)

    mesh = plsc.VectorSubcoreMesh(core_axis_name="c", subcore_axis_name="s",
                                  num_cores=2, num_subcores=16)
    gath = pl.kernel(
        _sc_gather,
        out_type=jax.ShapeDtypeStruct((11, _P), jnp.float32),
        mesh=mesh,
        scratch_types=[pltpu.VMEM((_PW,), jnp.int32),
                       pltpu.VMEM((_PW,), jnp.float32),
                       pltpu.SemaphoreType.DMA],
    )(idx_all, score_flat, lse_flat, cflat, bbfl, pdfl)

    shp = (_P // 128, 128)
    closs, bloss = pl.pallas_call(
        _loss_body,
        out_specs=[pl.BlockSpec(memory_space=pltpu.SMEM),
                   pl.BlockSpec(memory_space=pltpu.SMEM)],
        out_shape=[jax.ShapeDtypeStruct((1, 1), jnp.float32),
                   jax.ShapeDtypeStruct((1, 1), jnp.float32)],
    )(score0.reshape(_ROWS, 128), gath[0].reshape(shp),
      gath[1].reshape(shp), gath[2].reshape(shp),
      wv_p.reshape(shp), wp_p.reshape(shp),
      gath[3:7].reshape(4, *shp), gath[7:11].reshape(4, *shp),
      tb_p.T.reshape(4, *shp))

    return (closs[0, 0], bloss[0, 0])
